# baseline (speedup 1.0000x reference)
import jax
import jax.numpy as jnp
from jax import lax
from jax.experimental import pallas as pl
from jax.experimental.pallas import tpu as pltpu
from jax.experimental.pallas import tpu_sc as plsc

E_ENT = 100000
B = 128
NC = 2
NS = 16
NW = NC * NS
ROWS_PER_W = B // NW


def _sc_body(x_hbm, zrow_hbm, out_hbm, x_v, row_v):
    wid = lax.axis_index("s") * NC + lax.axis_index("c")
    pltpu.sync_copy(x_hbm, x_v)
    pltpu.sync_copy(zrow_hbm, row_v)
    vec = x_v[pl.ds((wid // 4) * 16, 16)]
    lane_ids = lax.broadcasted_iota(jnp.int32, (16,), 0)
    ones16 = jnp.ones((16,), jnp.float32)
    zeros16 = jnp.zeros((16,), jnp.float32)
    for j in range(ROWS_PER_W):
        row = wid * ROWS_PER_W + j
        lane = (wid % 4) * ROWS_PER_W + j
        mask = lane_ids == lane
        plsc.store_scatter(row_v, [vec], ones16, mask=mask)
        pltpu.sync_copy(row_v, out_hbm.at[row])
        plsc.store_scatter(row_v, [vec], zeros16, mask=mask)


def kernel(input_x, input_r, e2triple, triple2e, r2triple, emb_table,
           W_ih, W_hh, b_ih, b_hh, W_lin, b_lin):
    x_i32 = input_x.astype(jnp.int32)
    zrow = jnp.zeros((E_ENT,), jnp.float32)
    sc = pl.kernel(
        _sc_body,
        out_type=jax.ShapeDtypeStruct((B, E_ENT), jnp.float32),
        mesh=plsc.VectorSubcoreMesh(core_axis_name="c", subcore_axis_name="s"),
        scratch_types=[
            pltpu.VMEM((B,), jnp.int32),
            pltpu.VMEM((E_ENT,), jnp.float32),
        ],
        compiler_params=pltpu.CompilerParams(needs_layout_passes=False),
    )
    return sc(x_i32, zrow)
